# Initial kernel scaffold; baseline (speedup 1.0000x reference)
import jax
import jax.numpy as jnp
from jax.experimental import pallas as pl


def kernel(objectness, box_regression):
    raise NotImplementedError("write your pallas kernel here")


# speedup vs baseline: 1.9190x; 1.9190x over previous
import jax
import jax.numpy as jnp
from jax.experimental import pallas as pl
from jax.experimental.pallas import tpu as pltpu

_PRE_N = 2000
_P = 2048
_B = 256
_POST_N = 1000
_NMS_THRESH = 0.7
_MIN_SIZE = 4.0
_BASE = 640.0
_SCALE = 0.25
_SCORE_THRESH = 0.1
_NMS_ITERS = 20


def _nms_body(breg_ref, idx_ref, score_ref, boxes_ref, final_ref, c_ref):
    W = 256
    idx = idx_ref[0]
    score = score_ref[0]

    gx = (jnp.astype(idx & (W - 1), jnp.float32) + 0.5) / _SCALE
    gy = (jnp.astype(idx >> 8, jnp.float32) + 0.5) / _SCALE

    d0 = breg_ref[0, 0:1, :] * _BASE
    d1 = breg_ref[0, 1:2, :] * _BASE
    d2 = breg_ref[0, 2:3, :] * _BASE
    d3 = breg_ref[0, 3:4, :] * _BASE
    ang = breg_ref[0, 4:5, :]

    cx = gx + (d1 - d3) * 0.5
    cy = gy + (d2 - d0) * 0.5
    w = d1 + d3
    h = d0 + d2

    img_w = W / _SCALE
    cx = jnp.clip(cx, 0.0, img_w - 1.0)
    cy = jnp.clip(cy, 0.0, img_w - 1.0)

    valid = (score > _SCORE_THRESH) & (w >= _MIN_SIZE) & (h >= _MIN_SIZE)

    x1 = cx - w * 0.5
    y1 = cy - h * 0.5
    x2 = cx + w * 0.5
    y2 = cy + h * 0.5
    area = (x2 - x1) * (y2 - y1)

    x1c = jnp.transpose(x1)
    y1c = jnp.transpose(y1)
    x2c = jnp.transpose(x2)
    y2c = jnp.transpose(y2)
    areac = jnp.transpose(area)

    for k in range(_P // _B):
        lo, hi = k * _B, (k + 1) * _B
        bx1 = x1c[lo:hi, :]
        by1 = y1c[lo:hi, :]
        bx2 = x2c[lo:hi, :]
        by2 = y2c[lo:hi, :]
        barea = areac[lo:hi, :]
        iw = jnp.maximum(jnp.minimum(bx2, x2) - jnp.maximum(bx1, x1), 0.0)
        ih = jnp.maximum(jnp.minimum(by2, y2) - jnp.maximum(by1, y1), 0.0)
        inter = iw * ih
        iou = inter / (barea + area - inter + 1e-9)
        gi = k * _B + jax.lax.broadcasted_iota(jnp.int32, (_B, _P), 0)
        gj = jax.lax.broadcasted_iota(jnp.int32, (_B, _P), 1)
        c_ref[lo:hi, :] = jnp.where(gi < gj, iou, 0.0)

    keep_col = jnp.transpose(valid.astype(jnp.float32))
    keep_row = valid
    for _ in range(_NMS_ITERS):
        max_c = jnp.max(c_ref[...] * keep_col, axis=0, keepdims=True)
        keep_row = (max_c <= _NMS_THRESH) & valid
        keep_col = jnp.transpose(keep_row.astype(jnp.float32))

    boxes_ref[0, 0:1, :] = cx
    boxes_ref[0, 1:2, :] = cy
    boxes_ref[0, 2:3, :] = w
    boxes_ref[0, 3:4, :] = h
    boxes_ref[0, 4:5, :] = ang
    final_ref[0] = jnp.where(keep_row, score, -jnp.inf)


def kernel(objectness, box_regression):
    N, A, H, W = objectness.shape
    obj = jnp.transpose(objectness, (0, 2, 3, 1)).reshape(N, -1)
    breg_flat = box_regression.reshape(N, 5, H * W)

    scores, idx = jax.lax.top_k(obj, _PRE_N)
    gathered = jnp.take_along_axis(breg_flat, idx[:, None, :], axis=2)

    pad = _P - _PRE_N
    breg_p = jnp.pad(gathered, ((0, 0), (0, 0), (0, pad)))
    idx_p = jnp.pad(idx.astype(jnp.int32), ((0, 0), (0, pad)))[:, None, :]
    score_p = jnp.pad(scores, ((0, 0), (0, pad)), constant_values=-jnp.inf)[:, None, :]

    boxes, final = pl.pallas_call(
        _nms_body,
        grid=(N,),
        in_specs=[
            pl.BlockSpec((1, 5, _P), lambda n: (n, 0, 0)),
            pl.BlockSpec((1, 1, _P), lambda n: (n, 0, 0)),
            pl.BlockSpec((1, 1, _P), lambda n: (n, 0, 0)),
        ],
        out_specs=[
            pl.BlockSpec((1, 5, _P), lambda n: (n, 0, 0)),
            pl.BlockSpec((1, 1, _P), lambda n: (n, 0, 0)),
        ],
        out_shape=[
            jax.ShapeDtypeStruct((N, 5, _P), jnp.float32),
            jax.ShapeDtypeStruct((N, 1, _P), jnp.float32),
        ],
        scratch_shapes=[pltpu.VMEM((_P, _P), jnp.float32)],
    )(breg_p, idx_p, score_p)

    top_s, top_i = jax.lax.top_k(final[:, 0, :], _POST_N)
    boxes_t = jnp.transpose(boxes, (0, 2, 1))
    top_b = jnp.take_along_axis(boxes_t, top_i[:, :, None], axis=1)
    fin = jnp.isfinite(top_s)
    top_s = jnp.where(fin, top_s, 0.0)
    top_b = jnp.where(fin[:, :, None], top_b, 0.0)
    return jnp.concatenate([top_b, top_s[:, :, None]], axis=2)
